# baseline (speedup 1.0000x reference)
import jax
import jax.numpy as jnp
from jax.experimental import pallas as pl
from jax.experimental.pallas import tpu as pltpu

_TV = 512
_KSPLIT = 4


def _temp_body(x_ref, wl_ref, bl_ref, rt_ref):
    tl = jnp.sum(x_ref[...] * wl_ref[...], axis=1, keepdims=True) + bl_ref[...]
    rt = 1.0 / jax.nn.sigmoid(tl)
    rt_ref[...] = jnp.broadcast_to(rt, rt_ref.shape)


def _main_body(xb_ref, rt_ref, *rest):
    w_refs = rest[:_KSPLIT]
    b_ref, o_ref = rest[_KSPLIT], rest[_KSPLIT + 1]
    kc = xb_ref.shape[1] // _KSPLIT
    acc = jnp.dot(xb_ref[:, 0:kc], w_refs[0][...],
                  preferred_element_type=jnp.float32)
    for i in range(1, _KSPLIT):
        acc += jnp.dot(xb_ref[:, i * kc:(i + 1) * kc], w_refs[i][...],
                       preferred_element_type=jnp.float32)
    o_ref[...] = (acc + b_ref[...]) * rt_ref[...][:, 0:1]


def kernel(inputs, W, b):
    x = inputs
    n, k = x.shape
    v = W.shape[1] - 1
    kc = k // _KSPLIT

    xb = x.astype(jnp.bfloat16)
    wl = W[:, -1].reshape(1, k)
    bl = b[-1].reshape(1, 1)
    b2 = b[:-1].reshape(1, v)

    rt = pl.pallas_call(
        _temp_body,
        out_shape=jax.ShapeDtypeStruct((n, 128), jnp.float32),
    )(x, wl, bl)

    w_specs = [
        pl.BlockSpec((kc, _TV), lambda j, i=i: (i, j)) for i in range(_KSPLIT)
    ]
    out = pl.pallas_call(
        _main_body,
        grid=(pl.cdiv(v, _TV),),
        in_specs=[
            pl.BlockSpec((n, k), lambda j: (0, 0)),
            pl.BlockSpec((n, 128), lambda j: (0, 0)),
            *w_specs,
            pl.BlockSpec((1, _TV), lambda j: (0, j)),
        ],
        out_specs=pl.BlockSpec((n, _TV), lambda j: (0, j)),
        out_shape=jax.ShapeDtypeStruct((n, v), jnp.float32),
        compiler_params=pltpu.CompilerParams(
            dimension_semantics=("parallel",)),
    )(xb, rt, *([W] * _KSPLIT), b2)
    return out


# speedup vs baseline: 2.7485x; 2.7485x over previous
import jax
import jax.numpy as jnp
from jax.experimental import pallas as pl
from jax.experimental.pallas import tpu as pltpu

_TV = 512


def _temp_body(wl_ref, xt_ref, bl_ref, rt_ref):
    tl = jnp.dot(wl_ref[...], xt_ref[...],
                 preferred_element_type=jnp.float32) + bl_ref[...]
    rt = 1.0 / jax.nn.sigmoid(tl)
    rt_ref[...] = jnp.broadcast_to(rt, rt_ref.shape)


def _main_body(wt_ref, xtb_ref, rt_ref, b_ref, o_ref):
    acc = jnp.dot(wt_ref[...], xtb_ref[...],
                  preferred_element_type=jnp.float32)
    o_ref[...] = (acc + b_ref[...]) * rt_ref[0:1, :]


def kernel(inputs, W, b):
    x = inputs
    n, k = x.shape
    v = W.shape[1] - 1

    wt = W.T
    xt = x.T
    xtb = xt.astype(jnp.bfloat16)
    wl = wt[v:v + 1, :]
    bl = b[-1].reshape(1, 1)
    b2 = b[:-1].reshape(v, 1)

    rt = pl.pallas_call(
        _temp_body,
        out_shape=jax.ShapeDtypeStruct((8, n), jnp.float32),
    )(wl, xt, bl)

    out_t = pl.pallas_call(
        _main_body,
        grid=(pl.cdiv(v, _TV),),
        in_specs=[
            pl.BlockSpec((_TV, k), lambda j: (j, 0)),
            pl.BlockSpec((k, n), lambda j: (0, 0)),
            pl.BlockSpec((8, n), lambda j: (0, 0)),
            pl.BlockSpec((_TV, 1), lambda j: (j, 0)),
        ],
        out_specs=pl.BlockSpec((_TV, n), lambda j: (j, 0)),
        out_shape=jax.ShapeDtypeStruct((v, n), jnp.float32),
        compiler_params=pltpu.CompilerParams(
            dimension_semantics=("parallel",)),
    )(wt, xtb, rt, b2)
    return out_t.T


# speedup vs baseline: 2.8157x; 1.0245x over previous
import jax
import jax.numpy as jnp
from jax.experimental import pallas as pl
from jax.experimental.pallas import tpu as pltpu

_TV = 1024


def _temp_body(wl_ref, xt_ref, bl_ref, rt_ref):
    tl = jnp.dot(wl_ref[...], xt_ref[...],
                 preferred_element_type=jnp.float32) + bl_ref[...]
    rt = 1.0 / jax.nn.sigmoid(tl)
    rt_ref[...] = jnp.broadcast_to(rt, rt_ref.shape)


def _main_body(wt_ref, xtb_ref, rt_ref, b_ref, o_ref):
    acc = jnp.dot(wt_ref[...], xtb_ref[...],
                  preferred_element_type=jnp.float32)
    o_ref[...] = (acc + b_ref[...]) * rt_ref[0:1, :]


def kernel(inputs, W, b):
    x = inputs
    n, k = x.shape
    v = W.shape[1] - 1

    wt = W.T
    xt = x.T
    xtb = xt.astype(jnp.bfloat16)
    wl = wt[v:v + 1, :]
    bl = b[-1].reshape(1, 1)
    b2 = b[:-1].reshape(v, 1)

    rt = pl.pallas_call(
        _temp_body,
        out_shape=jax.ShapeDtypeStruct((8, n), jnp.float32),
    )(wl, xt, bl)

    out_t = pl.pallas_call(
        _main_body,
        grid=(pl.cdiv(v, _TV),),
        in_specs=[
            pl.BlockSpec((_TV, k), lambda j: (j, 0)),
            pl.BlockSpec((k, n), lambda j: (0, 0)),
            pl.BlockSpec((8, n), lambda j: (0, 0)),
            pl.BlockSpec((_TV, 1), lambda j: (j, 0)),
        ],
        out_specs=pl.BlockSpec((_TV, n), lambda j: (j, 0)),
        out_shape=jax.ShapeDtypeStruct((v, n), jnp.float32),
        compiler_params=pltpu.CompilerParams(
            dimension_semantics=("parallel",)),
    )(wt, xtb, rt, b2)
    return out_t.T
